# Initial kernel scaffold; baseline (speedup 1.0000x reference)
import jax
import jax.numpy as jnp
from jax.experimental import pallas as pl


def kernel(x, edge_index, edge_weight, W_weight, W_bias):
    raise NotImplementedError("write your pallas kernel here")


# speedup vs baseline: 6.4028x; 6.4028x over previous
import functools

import jax
import jax.numpy as jnp
from jax import lax
from jax.experimental import pallas as pl
from jax.experimental.pallas import tpu as pltpu
from jax.experimental.pallas import tpu_sc as plsc

N_NODES = 10000
PAD_NODES = 10240
D = 128
NC = 2
NS = 16
NW = NC * NS
CHUNK = 80
LANES = 16


def _sc_spmm_body(x_hbm, src_hbm, dst_hbm, w_hbm, zeros_hbm, out_hbm,
                  src_v, dst_v, w_v, rows_v, acc_sh):
    c = lax.axis_index("c")
    s = lax.axis_index("s")
    wid = c * NS + s
    e_per_tile = src_v.shape[0]
    n_chunks = dst_v.shape[0]
    rows_per_tile = PAD_NODES // NS

    base = wid * e_per_tile
    pltpu.sync_copy(src_hbm.at[pl.ds(base, e_per_tile)], src_v)
    pltpu.sync_copy(w_hbm.at[pl.ds(base, e_per_tile)], w_v)
    pltpu.sync_copy(dst_hbm.at[wid], dst_v)
    pltpu.sync_copy(zeros_hbm.at[pl.ds(s * rows_per_tile, rows_per_tile)],
                    acc_sh.at[pl.ds(s * rows_per_tile, rows_per_tile)])
    plsc.subcore_barrier()

    def chunk_body(i, carry):
        pltpu.sync_copy(x_hbm.at[src_v.at[pl.ds(i * CHUNK, CHUNK)]], rows_v)

        def grp_body(g, carry2):
            w16 = w_v[pl.ds(i * CHUNK + g * LANES, LANES)]
            for j in range(LANES):
                wj = jnp.broadcast_to(w16[j], (LANES,))
                row = g * LANES + j
                for v in range(D // LANES):
                    sl = pl.ds(v * LANES, LANES)
                    rows_v[row, sl] = rows_v[row, sl] * wj
            return carry2

        lax.fori_loop(0, CHUNK // LANES, grp_body, 0)
        pltpu.sync_copy(rows_v, acc_sh.at[dst_v.at[i]], add=True)
        return carry

    lax.fori_loop(0, n_chunks, chunk_body, 0)
    plsc.subcore_barrier()
    pltpu.sync_copy(acc_sh.at[pl.ds(s * rows_per_tile, rows_per_tile)],
                    out_hbm.at[c, pl.ds(s * rows_per_tile, rows_per_tile)])


def _sc_spmm(x, src, dst2d, w, zeros):
    n_edges = src.shape[0]
    e_per_tile = n_edges // NW
    n_chunks = e_per_tile // CHUNK
    mesh = plsc.VectorSubcoreMesh(core_axis_name="c", subcore_axis_name="s")
    return pl.kernel(
        _sc_spmm_body,
        out_type=jax.ShapeDtypeStruct((NC, PAD_NODES, D), jnp.float32),
        mesh=mesh,
        scratch_types=[
            pltpu.VMEM((e_per_tile,), jnp.int32),
            pltpu.VMEM((n_chunks, CHUNK), jnp.int32),
            pltpu.VMEM((e_per_tile,), jnp.float32),
            pltpu.VMEM((CHUNK, D), jnp.float32),
            pltpu.VMEM_SHARED((PAD_NODES, D), jnp.float32),
        ],
    )(x, src, dst2d, w, zeros)


def _tc_linear_body(a0_ref, a1_ref, w_ref, b_ref, o_ref):
    a = a0_ref[...] + a1_ref[...]
    o_ref[...] = lax.dot_general(
        a, w_ref[...], (((1,), (1,)), ((), ())),
        preferred_element_type=jnp.float32) + b_ref[...]


def _tc_linear(a0, a1, W, b2d):
    blk = 400
    grid = N_NODES // blk
    return pl.pallas_call(
        _tc_linear_body,
        grid=(grid,),
        in_specs=[
            pl.BlockSpec((blk, D), lambda i: (i, 0)),
            pl.BlockSpec((blk, D), lambda i: (i, 0)),
            pl.BlockSpec((D, D), lambda i: (0, 0)),
            pl.BlockSpec((1, D), lambda i: (0, 0)),
        ],
        out_specs=pl.BlockSpec((blk, D), lambda i: (i, 0)),
        out_shape=jax.ShapeDtypeStruct((N_NODES, D), jnp.float32),
    )(a0, a1, W, b2d)


def kernel(x, edge_index, edge_weight, W_weight, W_bias):
    n_edges = edge_index.shape[1]
    src = edge_index[0].astype(jnp.int32)
    dst = edge_index[1].astype(jnp.int32)
    dst3d = dst.reshape(NW, n_edges // (NW * CHUNK), CHUNK)
    zeros = jnp.zeros((PAD_NODES, D), jnp.float32)
    parts = _sc_spmm(x, src, dst3d, edge_weight, zeros)
    return _tc_linear(parts[0], parts[1], W_weight, W_bias.reshape(1, D))


# speedup vs baseline: 8.4774x; 1.3240x over previous
import jax
import jax.numpy as jnp
from jax import lax
from jax.experimental import pallas as pl
from jax.experimental.pallas import tpu as pltpu
from jax.experimental.pallas import tpu_sc as plsc

N_NODES = 10000
PAD_NODES = 10240
D = 128
NC = 2
NS = 16
NW = NC * NS
CHUNK = 80
LANES = 16
SRC, DST = 0, 1


def _scale_rows(rows_ref, w_ref):
    def grp_body(g, carry):
        w16 = w_ref[0, pl.ds(g * LANES, LANES)]
        for j in range(LANES):
            wj = jnp.broadcast_to(w16[j], (LANES,))
            row = g * LANES + j
            for v in range(D // LANES):
                sl = pl.ds(v * LANES, LANES)
                rows_ref[row, sl] = rows_ref[row, sl] * wj
        return carry

    lax.fori_loop(0, CHUNK // LANES, grp_body, 0)


def _sc_spmm_body(x_hbm, meta_hbm, w_hbm, zeros_hbm, out_hbm,
                  meta_a, meta_b, w_a, w_b, rows_a, rows_b, acc_sh,
                  sem_ma, sem_mb, sem_ga, sem_gb):
    c = lax.axis_index("c")
    s = lax.axis_index("s")
    wid = c * NS + s
    n_chunks = meta_hbm.shape[1]
    rows_per_tile = PAD_NODES // NS

    pltpu.sync_copy(zeros_hbm.at[pl.ds(s * rows_per_tile, rows_per_tile)],
                    acc_sh.at[pl.ds(s * rows_per_tile, rows_per_tile)])
    plsc.subcore_barrier()

    def start_meta(i, meta_ref, w_ref, sem):
        pltpu.async_copy(meta_hbm.at[wid, i], meta_ref, sem)
        pltpu.async_copy(w_hbm.at[wid, i], w_ref, sem)

    def wait_meta(i, meta_ref, w_ref, sem):
        pltpu.make_async_copy(meta_hbm.at[wid, i], meta_ref, sem).wait()
        pltpu.make_async_copy(w_hbm.at[wid, i], w_ref, sem).wait()

    def start_gather(meta_ref, rows_ref, sem):
        pltpu.async_copy(x_hbm.at[meta_ref.at[SRC]], rows_ref, sem)

    def wait_gather(meta_ref, rows_ref, sem):
        pltpu.make_async_copy(x_hbm.at[meta_ref.at[SRC]], rows_ref, sem).wait()

    def finish_chunk(meta_ref, w_ref, rows_ref):
        _scale_rows(rows_ref, w_ref)
        pltpu.sync_copy(rows_ref, acc_sh.at[meta_ref.at[DST]], add=True)

    start_meta(0, meta_a, w_a, sem_ma)
    start_meta(1, meta_b, w_b, sem_mb)
    wait_meta(0, meta_a, w_a, sem_ma)
    start_gather(meta_a, rows_a, sem_ga)

    def half_step(i, mp, mq, wp, wq, rp, rq, smp, smq, sgp, sgq, last):
        wait_gather(mp, rp, sgp)
        wait_meta(i + 1, mq, wq, smq)
        start_gather(mq, rq, sgq)
        finish_chunk(mp, wp, rp)
        if last:
            @pl.when(i + 2 < n_chunks)
            def _():
                start_meta(i + 2, mp, wp, smp)
        else:
            start_meta(i + 2, mp, wp, smp)

    def pair_body(k, carry):
        i = 2 * k
        half_step(i, meta_a, meta_b, w_a, w_b, rows_a, rows_b,
                  sem_ma, sem_mb, sem_ga, sem_gb, False)
        half_step(i + 1, meta_b, meta_a, w_b, w_a, rows_b, rows_a,
                  sem_mb, sem_ma, sem_gb, sem_ga, True)
        return carry

    lax.fori_loop(0, (n_chunks - 1) // 2, pair_body, 0)
    wait_gather(meta_a, rows_a, sem_ga)
    finish_chunk(meta_a, w_a, rows_a)

    plsc.subcore_barrier()
    pltpu.sync_copy(acc_sh.at[pl.ds(s * rows_per_tile, rows_per_tile)],
                    out_hbm.at[c, pl.ds(s * rows_per_tile, rows_per_tile)])


def _sc_spmm(x, meta, w4, zeros):
    n_chunks = meta.shape[1]
    mesh = plsc.VectorSubcoreMesh(core_axis_name="c", subcore_axis_name="s")
    return pl.kernel(
        _sc_spmm_body,
        out_type=jax.ShapeDtypeStruct((NC, PAD_NODES, D), jnp.float32),
        mesh=mesh,
        scratch_types=[
            pltpu.VMEM((2, CHUNK), jnp.int32),
            pltpu.VMEM((2, CHUNK), jnp.int32),
            pltpu.VMEM((1, CHUNK), jnp.float32),
            pltpu.VMEM((1, CHUNK), jnp.float32),
            pltpu.VMEM((CHUNK, D), jnp.float32),
            pltpu.VMEM((CHUNK, D), jnp.float32),
            pltpu.VMEM_SHARED((PAD_NODES, D), jnp.float32),
            pltpu.SemaphoreType.DMA,
            pltpu.SemaphoreType.DMA,
            pltpu.SemaphoreType.DMA,
            pltpu.SemaphoreType.DMA,
        ],
    )(x, meta, w4, zeros)


def _tc_linear_body(p_ref, w_ref, b_ref, o_ref):
    a = p_ref[0] + p_ref[1]
    o_ref[...] = lax.dot_general(
        a, w_ref[...], (((1,), (1,)), ((), ())),
        preferred_element_type=jnp.float32) + b_ref[...]


def _tc_linear(parts, W, b2d):
    blk = 400
    grid = N_NODES // blk
    return pl.pallas_call(
        _tc_linear_body,
        grid=(grid,),
        in_specs=[
            pl.BlockSpec((NC, blk, D), lambda i: (0, i, 0)),
            pl.BlockSpec((D, D), lambda i: (0, 0)),
            pl.BlockSpec((1, D), lambda i: (0, 0)),
        ],
        out_specs=pl.BlockSpec((blk, D), lambda i: (i, 0)),
        out_shape=jax.ShapeDtypeStruct((N_NODES, D), jnp.float32),
    )(parts, W, b2d)


def kernel(x, edge_index, edge_weight, W_weight, W_bias):
    n_edges = edge_index.shape[1]
    src = edge_index[0].astype(jnp.int32)
    dst = edge_index[1].astype(jnp.int32)
    n_chunks = n_edges // (NW * CHUNK)
    meta = jnp.stack([src, dst])
    meta = meta.reshape(2, NW, n_chunks, CHUNK).transpose(1, 2, 0, 3)
    w4 = edge_weight.reshape(NW, n_chunks, 1, CHUNK)
    zeros = jnp.zeros((PAD_NODES, D), jnp.float32)
    parts = _sc_spmm(x, meta, w4, zeros)
    return _tc_linear(parts, W_weight, W_bias.reshape(1, D))


# speedup vs baseline: 10.5644x; 1.2462x over previous
import jax
import jax.numpy as jnp
from jax import lax
from jax.experimental import pallas as pl
from jax.experimental.pallas import tpu as pltpu
from jax.experimental.pallas import tpu_sc as plsc

N_NODES = 10000
PAD_NODES = 10240
D = 128
NC = 2
NS = 16
NW = NC * NS
CHUNK = 80
LANES = 16
NMETA = 4


def _scale_rows(rows_ref, wb, m, i):
    del i

    def grp_body(g, carry):
        w16 = wb[m, 0, pl.ds(g * LANES, LANES)]
        for j in range(LANES):
            wj = jnp.broadcast_to(w16[j], (LANES,))
            row = g * LANES + j
            for v in range(D // LANES):
                sl = pl.ds(v * LANES, LANES)
                rows_ref[row, sl] = rows_ref[row, sl] * wj
        return carry

    lax.fori_loop(0, CHUNK // LANES, grp_body, 0)


def _sc_spmm_body(x_hbm, ei_hbm, w_hbm, out_hbm,
                  srcb, dstb, wb, rows, acc_sh,
                  sem_m, sem_g, sem_s):
    c = lax.axis_index("c")
    s = lax.axis_index("s")
    wid = c * NS + s
    n_chunks = ei_hbm.shape[2]
    rows_per_tile = PAD_NODES // NS

    zv = jnp.zeros((LANES,), jnp.float32)

    def zero_body(j, carry):
        for v in range(D // LANES):
            rows[0, j, pl.ds(v * LANES, LANES)] = zv
        return carry

    lax.fori_loop(0, CHUNK, zero_body, 0)
    for q in range(rows_per_tile // CHUNK):
        pltpu.sync_copy(rows.at[0],
                        acc_sh.at[pl.ds(s * rows_per_tile + q * CHUNK, CHUNK)])
    plsc.subcore_barrier()

    def start_meta(i, m):
        pltpu.async_copy(ei_hbm.at[0, wid, i], srcb.at[m], sem_m.at[m])
        pltpu.async_copy(ei_hbm.at[1, wid, i], dstb.at[m], sem_m.at[m])
        pltpu.async_copy(w_hbm.at[wid, i], wb.at[m], sem_m.at[m])

    def wait_meta(i, m):
        pltpu.make_async_copy(ei_hbm.at[0, wid, i], srcb.at[m],
                              sem_m.at[m]).wait()
        pltpu.make_async_copy(ei_hbm.at[1, wid, i], dstb.at[m],
                              sem_m.at[m]).wait()
        pltpu.make_async_copy(w_hbm.at[wid, i], wb.at[m], sem_m.at[m]).wait()

    def start_gather(m, p):
        pltpu.async_copy(x_hbm.at[srcb.at[m, 0]], rows.at[p], sem_g.at[p])

    def wait_gather(m, p):
        pltpu.make_async_copy(x_hbm.at[srcb.at[m, 0]], rows.at[p],
                              sem_g.at[p]).wait()

    def start_scatter(m, p):
        pltpu.async_copy(rows.at[p], acc_sh.at[dstb.at[m, 0]], sem_s.at[p],
                         add=True)

    def wait_scatter(m, p):
        pltpu.make_async_copy(rows.at[p], acc_sh.at[dstb.at[m, 0]],
                              sem_s.at[p]).wait()

    for j in range(NMETA - 1):
        start_meta(j, j)
    wait_meta(0, 0)
    start_gather(0, 0)
    wait_meta(1, 1)
    start_gather(1, 1)
    start_meta(NMETA - 1, NMETA - 1)
    wait_gather(0, 0)
    _scale_rows(rows.at[0], wb, 0, 0)
    start_scatter(0, 0)

    def quad_body(k, carry):
        for o in range(1, 5):
            i = 4 * k + o
            m = o % NMETA
            p = o % 2
            wait_scatter((o - 1) % NMETA, 1 - p)

            @pl.when(i + 3 < n_chunks)
            def _():
                start_meta(i + 3, (o + 3) % NMETA)

            @pl.when(i + 1 < n_chunks)
            def _():
                wait_meta(i + 1, (o + 1) % NMETA)
                start_gather((o + 1) % NMETA, 1 - p)

            wait_gather(m, p)
            _scale_rows(rows.at[p], wb, m, i)
            start_scatter(m, p)
        return carry

    lax.fori_loop(0, (n_chunks - 1) // 4, quad_body, 0)
    wait_scatter((n_chunks - 1) % NMETA, (n_chunks - 1) % 2)

    plsc.subcore_barrier()
    pltpu.sync_copy(acc_sh.at[pl.ds(s * rows_per_tile, rows_per_tile)],
                    out_hbm.at[c, pl.ds(s * rows_per_tile, rows_per_tile)])


def _sc_spmm(x, ei5, w4):
    mesh = plsc.VectorSubcoreMesh(core_axis_name="c", subcore_axis_name="s")
    return pl.kernel(
        _sc_spmm_body,
        out_type=jax.ShapeDtypeStruct((NC, PAD_NODES, D), jnp.float32),
        mesh=mesh,
        scratch_types=[
            pltpu.VMEM((NMETA, 1, CHUNK), jnp.int32),
            pltpu.VMEM((NMETA, 1, CHUNK), jnp.int32),
            pltpu.VMEM((NMETA, 1, CHUNK), jnp.float32),
            pltpu.VMEM((2, CHUNK, D), jnp.float32),
            pltpu.VMEM_SHARED((PAD_NODES, D), jnp.float32),
            pltpu.SemaphoreType.DMA((NMETA,)),
            pltpu.SemaphoreType.DMA((2,)),
            pltpu.SemaphoreType.DMA((2,)),
        ],
    )(x, ei5, w4)


def _tc_linear_body(p_ref, w_ref, b_ref, o_ref):
    a = p_ref[0] + p_ref[1]
    o_ref[...] = lax.dot_general(
        a, w_ref[...], (((1,), (1,)), ((), ())),
        preferred_element_type=jnp.float32) + b_ref[...]


def _tc_linear(parts, W, b2d):
    blk = 400
    grid = N_NODES // blk
    return pl.pallas_call(
        _tc_linear_body,
        grid=(grid,),
        in_specs=[
            pl.BlockSpec((NC, blk, D), lambda i: (0, i, 0)),
            pl.BlockSpec((D, D), lambda i: (0, 0)),
            pl.BlockSpec((1, D), lambda i: (0, 0)),
        ],
        out_specs=pl.BlockSpec((blk, D), lambda i: (i, 0)),
        out_shape=jax.ShapeDtypeStruct((N_NODES, D), jnp.float32),
    )(parts, W, b2d)


def kernel(x, edge_index, edge_weight, W_weight, W_bias):
    n_edges = edge_index.shape[1]
    n_chunks = n_edges // (NW * CHUNK)
    ei5 = edge_index.astype(jnp.int32).reshape(2, NW, n_chunks, 1, CHUNK)
    w4 = edge_weight.reshape(NW, n_chunks, 1, CHUNK)
    parts = _sc_spmm(x, ei5, w4)
    return _tc_linear(parts, W_weight, W_bias.reshape(1, D))


# speedup vs baseline: 11.1427x; 1.0547x over previous
import jax
import jax.numpy as jnp
from jax import lax
from jax.experimental import pallas as pl
from jax.experimental.pallas import tpu as pltpu
from jax.experimental.pallas import tpu_sc as plsc

N_NODES = 10000
PAD_NODES = 10240
D = 128
NC = 2
NS = 16
NW = NC * NS
CHUNK = 80
LANES = 16
NMETA = 4


def _scale_rows(rows_ref, wb, m, i):
    del i

    def grp_body(g, carry):
        w16 = wb[m, 0, pl.ds(g * LANES, LANES)]
        for j in range(LANES):
            wj = jnp.broadcast_to(w16[j], (LANES,))
            row = g * LANES + j
            for v in range(D // LANES):
                sl = pl.ds(v * LANES, LANES)
                rows_ref[row, sl] = rows_ref[row, sl] * wj
        return carry

    lax.fori_loop(0, CHUNK // LANES, grp_body, 0)


def _sc_spmm_body(x_hbm, ei_hbm, w_hbm, out_hbm,
                  srcb, dstb, wb, rows, acc_sh,
                  sem_m, sem_g, sem_s):
    c = lax.axis_index("c")
    s = lax.axis_index("s")
    wid = c * NS + s
    n_chunks = ei_hbm.shape[2]
    rows_per_tile = PAD_NODES // NS

    zv = jnp.zeros((LANES,), jnp.float32)

    def zero_body(j, carry):
        for v in range(D // LANES):
            rows[0, j, pl.ds(v * LANES, LANES)] = zv
        return carry

    lax.fori_loop(0, CHUNK, zero_body, 0)
    for q in range(rows_per_tile // CHUNK):
        pltpu.sync_copy(rows.at[0],
                        acc_sh.at[pl.ds(s * rows_per_tile + q * CHUNK, CHUNK)])
    plsc.subcore_barrier()

    def start_meta(i, m):
        pltpu.async_copy(ei_hbm.at[0, wid, i], srcb.at[m], sem_m.at[m])
        pltpu.async_copy(ei_hbm.at[1, wid, i], dstb.at[m], sem_m.at[m])
        pltpu.async_copy(w_hbm.at[wid, i], wb.at[m], sem_m.at[m])

    def wait_meta(i, m):
        pltpu.make_async_copy(ei_hbm.at[0, wid, i], srcb.at[m],
                              sem_m.at[m]).wait()
        pltpu.make_async_copy(ei_hbm.at[1, wid, i], dstb.at[m],
                              sem_m.at[m]).wait()
        pltpu.make_async_copy(w_hbm.at[wid, i], wb.at[m], sem_m.at[m]).wait()

    def start_gather(m, p):
        pltpu.async_copy(x_hbm.at[srcb.at[m, 0]], rows.at[p], sem_g.at[p])

    def wait_gather(m, p):
        pltpu.make_async_copy(x_hbm.at[srcb.at[m, 0]], rows.at[p],
                              sem_g.at[p]).wait()

    def start_scatter(m, p):
        pltpu.async_copy(rows.at[p], acc_sh.at[dstb.at[m, 0]], sem_s.at[p],
                         add=True)

    def wait_scatter(m, p):
        pltpu.make_async_copy(rows.at[p], acc_sh.at[dstb.at[m, 0]],
                              sem_s.at[p]).wait()

    for j in range(NMETA - 1):
        start_meta(j, j)
    wait_meta(0, 0)
    start_gather(0, 0)
    wait_meta(1, 1)
    start_gather(1, 1)
    start_meta(NMETA - 1, NMETA - 1)
    wait_gather(0, 0)
    _scale_rows(rows.at[0], wb, 0, 0)
    start_scatter(0, 0)

    def quad_body(k, carry):
        for o in range(1, 5):
            i = 4 * k + o
            m = o % NMETA
            p = o % 2
            wait_scatter((o - 1) % NMETA, 1 - p)

            @pl.when(i + 3 < n_chunks)
            def _():
                start_meta(i + 3, (o + 3) % NMETA)

            @pl.when(i + 1 < n_chunks)
            def _():
                wait_meta(i + 1, (o + 1) % NMETA)
                start_gather((o + 1) % NMETA, 1 - p)

            wait_gather(m, p)
            _scale_rows(rows.at[p], wb, m, i)
            start_scatter(m, p)
        return carry

    lax.fori_loop(0, (n_chunks - 1) // 4, quad_body, 0)
    wait_scatter((n_chunks - 1) % NMETA, (n_chunks - 1) % 2)

    plsc.subcore_barrier()
    pltpu.sync_copy(acc_sh.at[pl.ds(s * rows_per_tile, rows_per_tile)],
                    out_hbm.at[c, pl.ds(s * rows_per_tile, rows_per_tile)])


def _sc_spmm(x, ei5, w4):
    mesh = plsc.VectorSubcoreMesh(core_axis_name="c", subcore_axis_name="s")
    return pl.kernel(
        _sc_spmm_body,
        out_type=jax.ShapeDtypeStruct((NC, PAD_NODES, D), jnp.float32),
        mesh=mesh,
        scratch_types=[
            pltpu.VMEM((NMETA, 1, CHUNK), jnp.int32),
            pltpu.VMEM((NMETA, 1, CHUNK), jnp.int32),
            pltpu.VMEM((NMETA, 1, CHUNK), jnp.float32),
            pltpu.VMEM((2, CHUNK, D), jnp.float32),
            pltpu.VMEM_SHARED((PAD_NODES, D), jnp.float32),
            pltpu.SemaphoreType.DMA((NMETA,)),
            pltpu.SemaphoreType.DMA((2,)),
            pltpu.SemaphoreType.DMA((2,)),
        ],
    )(x, ei5, w4)


def _tc_linear_body(p_ref, w_ref, b_ref, o_ref):
    a = p_ref[0] + p_ref[1]
    o_ref[...] = lax.dot_general(
        a, w_ref[...], (((1,), (1,)), ((), ())),
        preferred_element_type=jnp.float32) + b_ref[...]


def _tc_linear(parts, W, b2d):
    blk = 2000
    grid = N_NODES // blk
    return pl.pallas_call(
        _tc_linear_body,
        grid=(grid,),
        in_specs=[
            pl.BlockSpec((NC, blk, D), lambda i: (0, i, 0)),
            pl.BlockSpec((D, D), lambda i: (0, 0)),
            pl.BlockSpec((1, D), lambda i: (0, 0)),
        ],
        out_specs=pl.BlockSpec((blk, D), lambda i: (i, 0)),
        out_shape=jax.ShapeDtypeStruct((N_NODES, D), jnp.float32),
    )(parts, W, b2d)


def kernel(x, edge_index, edge_weight, W_weight, W_bias):
    n_edges = edge_index.shape[1]
    n_chunks = n_edges // (NW * CHUNK)
    ei5 = edge_index.astype(jnp.int32).reshape(2, NW, n_chunks, 1, CHUNK)
    w4 = edge_weight.reshape(NW, n_chunks, 1, CHUNK)
    parts = _sc_spmm(x, ei5, w4)
    return _tc_linear(parts, W_weight, W_bias.reshape(1, D))
